# Initial kernel scaffold; baseline (speedup 1.0000x reference)
import jax
import jax.numpy as jnp
from jax.experimental import pallas as pl


def kernel(x, pos, edge_index, batch, W, b):
    raise NotImplementedError("write your pallas kernel here")


# speedup vs baseline: 1.1521x; 1.1521x over previous
import functools

import jax
import jax.numpy as jnp
from jax import lax
from jax.experimental import pallas as pl
from jax.experimental.pallas import tpu as pltpu
from jax.experimental.pallas import tpu_sc as plsc

_LANES = 16


def _tc_node_body(xT_ref, posT_ref, Wx_ref, Wp_ref, b_ref, uT_ref, vT_ref, p3T_ref):
    pz = posT_ref[0:1, :]
    phi = posT_ref[1:2, :]
    px = jnp.cos(phi)
    py = jnp.sin(phi)
    p3T_ref[...] = jnp.concatenate([px, py, pz], axis=0)
    Wp = Wp_ref[...]
    v = Wp[:, 0:1] * px + Wp[:, 1:2] * py + Wp[:, 2:3] * pz
    vT_ref[...] = v
    uT_ref[...] = (
        jnp.dot(Wx_ref[...], xT_ref[...], preferred_element_type=jnp.float32)
        + v
        + b_ref[...]
    )


def _node_transform(xT, posT, Wx, Wp, b2, NP, D, BN):
    grid = (NP // BN,)
    return pl.pallas_call(
        _tc_node_body,
        grid=grid,
        in_specs=[
            pl.BlockSpec((D, BN), lambda j: (0, j)),
            pl.BlockSpec((2, BN), lambda j: (0, j)),
            pl.BlockSpec((D, D), lambda j: (0, 0)),
            pl.BlockSpec((D, 3), lambda j: (0, 0)),
            pl.BlockSpec((D, 1), lambda j: (0, 0)),
        ],
        out_specs=[
            pl.BlockSpec((D, BN), lambda j: (0, j)),
            pl.BlockSpec((D, BN), lambda j: (0, j)),
            pl.BlockSpec((3, BN), lambda j: (0, j)),
        ],
        out_shape=[
            jax.ShapeDtypeStruct((D, NP), jnp.float32),
            jax.ShapeDtypeStruct((D, NP), jnp.float32),
            jax.ShapeDtypeStruct((3, NP), jnp.float32),
        ],
    )(xT, posT, Wx, Wp, b2)


def _make_sc_agg(D, NP, E, CH, NC, NS):
    NW = NC * NS
    FPT = D // NW
    mesh = plsc.VectorSubcoreMesh(
        core_axis_name="c", subcore_axis_name="s", num_cores=NC, num_subcores=NS
    )

    @functools.partial(
        pl.kernel,
        out_type=jax.ShapeDtypeStruct((D * NP,), jnp.float32),
        mesh=mesh,
        compiler_params=pltpu.CompilerParams(needs_layout_passes=False),
        scratch_types=[
            pltpu.VMEM((FPT * NP,), jnp.float32),
            pltpu.VMEM((FPT * NP,), jnp.float32),
            pltpu.VMEM((FPT * NP,), jnp.float32),
            pltpu.VMEM((CH,), jnp.int32),
            pltpu.VMEM((CH,), jnp.int32),
        ],
    )
    def agg(uT_hbm, vT_hbm, src_hbm, dst_hbm, out_hbm, u_v, acc_v, v_v, src_v, dst_v):
        cid = lax.axis_index("c")
        sid = lax.axis_index("s")
        wid = sid * NC + cid
        base = wid * (FPT * NP)
        pltpu.sync_copy(uT_hbm.at[pl.ds(base, FPT * NP)], u_v)
        pltpu.sync_copy(uT_hbm.at[pl.ds(base, FPT * NP)], acc_v)
        pltpu.sync_copy(vT_hbm.at[pl.ds(base, FPT * NP)], v_v)

        def chunk_body(ci, _):
            pltpu.sync_copy(src_hbm.at[pl.ds(ci * CH, CH)], src_v)
            pltpu.sync_copy(dst_hbm.at[pl.ds(ci * CH, CH)], dst_v)

            def vec_body(i, _):
                s16 = src_v[pl.ds(i * _LANES, _LANES)]
                d16 = dst_v[pl.ds(i * _LANES, _LANES)]
                for f in range(FPT):
                    idx_s = s16 + jnp.int32(f * NP)
                    idx_d = d16 + jnp.int32(f * NP)
                    g = plsc.load_gather(u_v, [idx_s])
                    a = plsc.load_gather(acc_v, [idx_d])
                    m = jnp.maximum(g, a)

                    def conflict_cond(carry):
                        m_, rb_ = carry
                        return jnp.any(m_ > rb_)

                    def conflict_body(carry):
                        m_, rb_ = carry
                        plsc.store_scatter(acc_v, [idx_d], m_, mask=m_ > rb_)
                        rb2 = plsc.load_gather(acc_v, [idx_d])
                        return jnp.maximum(m_, rb2), rb2

                    lax.while_loop(conflict_cond, conflict_body, (m, a))
                return 0

            lax.fori_loop(0, CH // _LANES, vec_body, 0)
            return 0

        lax.fori_loop(0, E // CH, chunk_body, 0)

        def sub_body(i, _):
            sl = pl.ds(i * _LANES, _LANES)
            acc_v[sl] = acc_v[sl] - v_v[sl]
            return 0

        lax.fori_loop(0, FPT * NP // _LANES, sub_body, 0)
        pltpu.sync_copy(acc_v, out_hbm.at[pl.ds(base, FPT * NP)])

    return agg


def kernel(x, pos, edge_index, batch, W, b):
    N, D = x.shape
    E = edge_index.shape[1]
    NC, NS = 2, 16
    NW = NC * NS
    assert D % NW == 0
    NP = -(-N // 256) * 256

    CH = 0
    for cand in range(2048, 15, -16):
        if E % cand == 0:
            CH = cand
            break
    assert CH > 0

    xT = jnp.pad(x.T, ((0, 0), (0, NP - N)))
    posT = jnp.pad(pos.T, ((0, 0), (0, NP - N)))
    Wx = W[:, :D]
    Wp = W[:, D:]
    b2 = b[:, None]

    uT, vT, p3T = _node_transform(xT, posT, Wx, Wp, b2, NP, D, 512)

    src = edge_index[0]
    dst = edge_index[1]
    agg = _make_sc_agg(D, NP, E, CH, NC, NS)
    outF = agg(uT.reshape(D * NP), vT.reshape(D * NP), src, dst)

    out = outF.reshape(D, NP)[:, :N].T
    pos3 = p3T[:, :N].T
    return (out, pos3, batch)


# speedup vs baseline: 3.4011x; 2.9522x over previous
import functools

import jax
import jax.numpy as jnp
from jax import lax
from jax.experimental import pallas as pl
from jax.experimental.pallas import tpu as pltpu
from jax.experimental.pallas import tpu_sc as plsc

_LANES = 16


def _tc_node_body(xT_ref, posT_ref, Wx_ref, Wp_ref, b_ref, uT_ref, vT_ref, p3T_ref):
    pz = posT_ref[0:1, :]
    phi = posT_ref[1:2, :]
    px = jnp.cos(phi)
    py = jnp.sin(phi)
    p3T_ref[...] = jnp.concatenate([px, py, pz], axis=0)
    Wp = Wp_ref[...]
    v = Wp[:, 0:1] * px + Wp[:, 1:2] * py + Wp[:, 2:3] * pz
    vT_ref[...] = v
    uT_ref[...] = (
        jnp.dot(Wx_ref[...], xT_ref[...], preferred_element_type=jnp.float32)
        + v
        + b_ref[...]
    )


def _node_transform(xT, posT, Wx, Wp, b2, NP, D, BN):
    grid = (NP // BN,)
    return pl.pallas_call(
        _tc_node_body,
        grid=grid,
        in_specs=[
            pl.BlockSpec((D, BN), lambda j: (0, j)),
            pl.BlockSpec((2, BN), lambda j: (0, j)),
            pl.BlockSpec((D, D), lambda j: (0, 0)),
            pl.BlockSpec((D, 3), lambda j: (0, 0)),
            pl.BlockSpec((D, 1), lambda j: (0, 0)),
        ],
        out_specs=[
            pl.BlockSpec((D, BN), lambda j: (0, j)),
            pl.BlockSpec((D, BN), lambda j: (0, j)),
            pl.BlockSpec((3, BN), lambda j: (0, j)),
        ],
        out_shape=[
            jax.ShapeDtypeStruct((D, NP), jnp.float32),
            jax.ShapeDtypeStruct((D, NP), jnp.float32),
            jax.ShapeDtypeStruct((3, NP), jnp.float32),
        ],
    )(xT, posT, Wx, Wp, b2)


def _make_sc_agg(D, NP, E, CH, NC, NS):
    NW = NC * NS
    FPT = D // NW
    mesh = plsc.VectorSubcoreMesh(
        core_axis_name="c", subcore_axis_name="s", num_cores=NC, num_subcores=NS
    )

    @functools.partial(
        pl.kernel,
        out_type=jax.ShapeDtypeStruct((D * NP,), jnp.float32),
        mesh=mesh,
        compiler_params=pltpu.CompilerParams(needs_layout_passes=False),
        scratch_types=[
            pltpu.VMEM((FPT * NP,), jnp.float32),
            pltpu.VMEM((FPT * NP,), jnp.float32),
            pltpu.VMEM((FPT * NP,), jnp.float32),
            pltpu.VMEM((CH,), jnp.int32),
            pltpu.VMEM((CH,), jnp.int32),
        ],
    )
    def agg(uT_hbm, vT_hbm, src_hbm, dst_hbm, out_hbm, u_v, acc_v, v_v, src_v, dst_v):
        cid = lax.axis_index("c")
        sid = lax.axis_index("s")
        wid = sid * NC + cid
        base = wid * (FPT * NP)
        pltpu.sync_copy(uT_hbm.at[pl.ds(base, FPT * NP)], u_v)
        pltpu.sync_copy(uT_hbm.at[pl.ds(base, FPT * NP)], acc_v)
        pltpu.sync_copy(vT_hbm.at[pl.ds(base, FPT * NP)], v_v)

        def chunk_body(ci, _):
            pltpu.sync_copy(src_hbm.at[pl.ds(ci * CH, CH)], src_v)
            pltpu.sync_copy(dst_hbm.at[pl.ds(ci * CH, CH)], dst_v)

            def vec_body(i, _):
                s16 = src_v[pl.ds(i * _LANES, _LANES)]
                d16 = dst_v[pl.ds(i * _LANES, _LANES)]
                idx_s = [s16 + jnp.int32(f * NP) for f in range(FPT)]
                idx_d = [d16 + jnp.int32(f * NP) for f in range(FPT)]
                g = [plsc.load_gather(u_v, [idx_s[f]]) for f in range(FPT)]
                a = [plsc.load_gather(acc_v, [idx_d[f]]) for f in range(FPT)]
                m = [jnp.maximum(g[f], a[f]) for f in range(FPT)]
                for f in range(FPT):
                    plsc.store_scatter(acc_v, [idx_d[f]], m[f])
                rb = [plsc.load_gather(acc_v, [idx_d[f]]) for f in range(FPT)]
                bad = m[0] > rb[0]
                for f in range(1, FPT):
                    bad = bad | (m[f] > rb[f])

                @pl.when(jnp.any(bad))
                def _fixup():
                    for f in range(FPT):

                        def conflict_cond(carry):
                            m_, rb_ = carry
                            return jnp.any(m_ > rb_)

                        def conflict_body(carry, f=f):
                            m_, rb_ = carry
                            plsc.store_scatter(
                                acc_v, [idx_d[f]], m_, mask=m_ > rb_
                            )
                            rb2 = plsc.load_gather(acc_v, [idx_d[f]])
                            return jnp.maximum(m_, rb2), rb2

                        lax.while_loop(conflict_cond, conflict_body, (m[f], rb[f]))
                return 0

            lax.fori_loop(0, CH // _LANES, vec_body, 0)
            return 0

        lax.fori_loop(0, E // CH, chunk_body, 0)

        def sub_body(i, _):
            sl = pl.ds(i * _LANES, _LANES)
            acc_v[sl] = acc_v[sl] - v_v[sl]
            return 0

        lax.fori_loop(0, FPT * NP // _LANES, sub_body, 0)
        pltpu.sync_copy(acc_v, out_hbm.at[pl.ds(base, FPT * NP)])

    return agg


def kernel(x, pos, edge_index, batch, W, b):
    N, D = x.shape
    E = edge_index.shape[1]
    NC, NS = 2, 16
    NW = NC * NS
    assert D % NW == 0
    NP = -(-N // 256) * 256

    CH = 0
    for cand in range(2048, 15, -16):
        if E % cand == 0:
            CH = cand
            break
    assert CH > 0

    xT = jnp.pad(x.T, ((0, 0), (0, NP - N)))
    posT = jnp.pad(pos.T, ((0, 0), (0, NP - N)))
    Wx = W[:, :D]
    Wp = W[:, D:]
    b2 = b[:, None]

    uT, vT, p3T = _node_transform(xT, posT, Wx, Wp, b2, NP, D, 512)

    src = edge_index[0]
    dst = edge_index[1]
    agg = _make_sc_agg(D, NP, E, CH, NC, NS)
    outF = agg(uT.reshape(D * NP), vT.reshape(D * NP), src, dst)

    out = outF.reshape(D, NP)[:, :N].T
    pos3 = p3T[:, :N].T
    return (out, pos3, batch)


# speedup vs baseline: 4.0572x; 1.1929x over previous
import functools

import jax
import jax.numpy as jnp
from jax import lax
from jax.experimental import pallas as pl
from jax.experimental.pallas import tpu as pltpu
from jax.experimental.pallas import tpu_sc as plsc

_LANES = 16


def _tc_node_body(xT_ref, posT_ref, Wx_ref, Wp_ref, b_ref, uT_ref, vT_ref, p3T_ref):
    pz = posT_ref[0:1, :]
    phi = posT_ref[1:2, :]
    px = jnp.cos(phi)
    py = jnp.sin(phi)
    p3T_ref[...] = jnp.concatenate([px, py, pz], axis=0)
    Wp = Wp_ref[...]
    v = Wp[:, 0:1] * px + Wp[:, 1:2] * py + Wp[:, 2:3] * pz
    vT_ref[...] = v
    uT_ref[...] = (
        jnp.dot(Wx_ref[...], xT_ref[...], preferred_element_type=jnp.float32)
        + v
        + b_ref[...]
    )


def _node_transform(xT, posT, Wx, Wp, b2, NP, D, BN):
    grid = (NP // BN,)
    return pl.pallas_call(
        _tc_node_body,
        grid=grid,
        in_specs=[
            pl.BlockSpec((D, BN), lambda j: (0, j)),
            pl.BlockSpec((2, BN), lambda j: (0, j)),
            pl.BlockSpec((D, D), lambda j: (0, 0)),
            pl.BlockSpec((D, 3), lambda j: (0, 0)),
            pl.BlockSpec((D, 1), lambda j: (0, 0)),
        ],
        out_specs=[
            pl.BlockSpec((D, BN), lambda j: (0, j)),
            pl.BlockSpec((D, BN), lambda j: (0, j)),
            pl.BlockSpec((3, BN), lambda j: (0, j)),
        ],
        out_shape=[
            jax.ShapeDtypeStruct((D, NP), jnp.float32),
            jax.ShapeDtypeStruct((D, NP), jnp.float32),
            jax.ShapeDtypeStruct((3, NP), jnp.float32),
        ],
    )(xT, posT, Wx, Wp, b2)


def _make_sc_agg(D, NP, E, CH, NC, NS):
    NW = NC * NS
    FPT = D // NW
    mesh = plsc.VectorSubcoreMesh(
        core_axis_name="c", subcore_axis_name="s", num_cores=NC, num_subcores=NS
    )

    @functools.partial(
        pl.kernel,
        out_type=jax.ShapeDtypeStruct((D * NP,), jnp.float32),
        mesh=mesh,
        compiler_params=pltpu.CompilerParams(needs_layout_passes=False),
        scratch_types=[
            pltpu.VMEM((FPT * NP,), jnp.float32),
            pltpu.VMEM((FPT * NP,), jnp.float32),
            pltpu.VMEM((FPT * NP,), jnp.float32),
            pltpu.VMEM((CH,), jnp.int32),
            pltpu.VMEM((CH,), jnp.int32),
            pltpu.VMEM((CH,), jnp.int32),
            pltpu.VMEM((CH,), jnp.int32),
            pltpu.SemaphoreType.DMA,
            pltpu.SemaphoreType.DMA,
            pltpu.SemaphoreType.DMA,
            pltpu.SemaphoreType.DMA,
        ],
    )
    def agg(
        uT_hbm,
        vT_hbm,
        src_hbm,
        dst_hbm,
        out_hbm,
        u_v,
        acc_v,
        v_v,
        src_a,
        src_b,
        dst_a,
        dst_b,
        sem_sa,
        sem_sb,
        sem_da,
        sem_db,
    ):
        cid = lax.axis_index("c")
        sid = lax.axis_index("s")
        wid = sid * NC + cid
        base = wid * (FPT * NP)
        pltpu.sync_copy(uT_hbm.at[pl.ds(base, FPT * NP)], u_v)
        pltpu.sync_copy(uT_hbm.at[pl.ds(base, FPT * NP)], acc_v)
        pltpu.sync_copy(vT_hbm.at[pl.ds(base, FPT * NP)], v_v)

        def process(src_v, dst_v):
            def vec_body(i, _):
                s16 = src_v[pl.ds(i * _LANES, _LANES)]
                d16 = dst_v[pl.ds(i * _LANES, _LANES)]
                idx_s = [s16 + jnp.int32(f * NP) for f in range(FPT)]
                idx_d = [d16 + jnp.int32(f * NP) for f in range(FPT)]
                g = [plsc.load_gather(u_v, [idx_s[f]]) for f in range(FPT)]
                a = [plsc.load_gather(acc_v, [idx_d[f]]) for f in range(FPT)]
                m = [jnp.maximum(g[f], a[f]) for f in range(FPT)]
                for f in range(FPT):
                    plsc.store_scatter(acc_v, [idx_d[f]], m[f])
                rb = [plsc.load_gather(acc_v, [idx_d[f]]) for f in range(FPT)]
                bad = m[0] > rb[0]
                for f in range(1, FPT):
                    bad = bad | (m[f] > rb[f])

                @pl.when(jnp.any(bad))
                def _fixup():
                    for f in range(FPT):

                        def conflict_cond(carry):
                            m_, rb_ = carry
                            return jnp.any(m_ > rb_)

                        def conflict_body(carry, f=f):
                            m_, rb_ = carry
                            plsc.store_scatter(
                                acc_v, [idx_d[f]], m_, mask=m_ > rb_
                            )
                            rb2 = plsc.load_gather(acc_v, [idx_d[f]])
                            return jnp.maximum(m_, rb2), rb2

                        lax.while_loop(conflict_cond, conflict_body, (m[f], rb[f]))
                return 0

            lax.fori_loop(0, CH // _LANES, vec_body, 0)

        nchunks = E // CH
        bufs = [(src_a, dst_a, sem_sa, sem_da), (src_b, dst_b, sem_sb, sem_db)]
        for bi, (sb, db, ss, sd) in enumerate(bufs):
            pltpu.async_copy(src_hbm.at[pl.ds(bi * CH, CH)], sb, ss)
            pltpu.async_copy(dst_hbm.at[pl.ds(bi * CH, CH)], db, sd)

        def outer_body(oi, _):
            for bi, (sb, db, ss, sd) in enumerate(bufs):
                ci = oi * 2 + bi
                pltpu.make_async_copy(src_hbm.at[pl.ds(ci * CH, CH)], sb, ss).wait()
                pltpu.make_async_copy(dst_hbm.at[pl.ds(ci * CH, CH)], db, sd).wait()
                process(sb, db)
                nci = lax.rem(ci + 2, nchunks)
                pltpu.async_copy(src_hbm.at[pl.ds(nci * CH, CH)], sb, ss)
                pltpu.async_copy(dst_hbm.at[pl.ds(nci * CH, CH)], db, sd)
            return 0

        lax.fori_loop(0, nchunks // 2, outer_body, 0)
        for bi, (sb, db, ss, sd) in enumerate(bufs):
            pltpu.make_async_copy(src_hbm.at[pl.ds(bi * CH, CH)], sb, ss).wait()
            pltpu.make_async_copy(dst_hbm.at[pl.ds(bi * CH, CH)], db, sd).wait()

        def sub_body(i, _):
            sl = pl.ds(i * _LANES, _LANES)
            acc_v[sl] = acc_v[sl] - v_v[sl]
            return 0

        lax.fori_loop(0, FPT * NP // _LANES, sub_body, 0)
        pltpu.sync_copy(acc_v, out_hbm.at[pl.ds(base, FPT * NP)])

    return agg


def kernel(x, pos, edge_index, batch, W, b):
    N, D = x.shape
    E = edge_index.shape[1]
    NC, NS = 2, 16
    NW = NC * NS
    assert D % NW == 0
    NP = -(-N // 256) * 256

    CH = 0
    for cand in range(2048, 15, -16):
        if E % cand == 0 and (E // cand) % 2 == 0:
            CH = cand
            break
    assert CH > 0

    xT = jnp.pad(x.T, ((0, 0), (0, NP - N)))
    posT = jnp.pad(pos.T, ((0, 0), (0, NP - N)))
    Wx = W[:, :D]
    Wp = W[:, D:]
    b2 = b[:, None]

    uT, vT, p3T = _node_transform(xT, posT, Wx, Wp, b2, NP, D, 512)

    src = edge_index[0]
    dst = edge_index[1]
    agg = _make_sc_agg(D, NP, E, CH, NC, NS)
    outF = agg(uT.reshape(D * NP), vT.reshape(D * NP), src, dst)

    out = outF.reshape(D, NP)[:, :N].T
    pos3 = p3T[:, :N].T
    return (out, pos3, batch)


# speedup vs baseline: 4.4277x; 1.0913x over previous
import functools

import jax
import jax.numpy as jnp
from jax import lax
from jax.experimental import pallas as pl
from jax.experimental.pallas import tpu as pltpu
from jax.experimental.pallas import tpu_sc as plsc

_LANES = 16


def _tc_node_body(xT_ref, posT_ref, Wx_ref, Wp_ref, b_ref, uP_ref, vT_ref, p3T_ref):
    pz = posT_ref[0:1, :]
    phi = posT_ref[1:2, :]
    px = jnp.cos(phi)
    py = jnp.sin(phi)
    p3T_ref[...] = jnp.concatenate([px, py, pz], axis=0)
    Wp = Wp_ref[...]
    v = Wp[:, 0:1] * px + Wp[:, 1:2] * py + Wp[:, 2:3] * pz
    vT_ref[...] = v
    u = (
        jnp.dot(Wx_ref[...], xT_ref[...], preferred_element_type=jnp.float32)
        + v
        + b_ref[...]
    )
    D = u.shape[0]
    H = D // 2
    lo = lax.bitcast_convert_type(u[:H].astype(jnp.bfloat16), jnp.uint16)
    hi = lax.bitcast_convert_type(u[H:].astype(jnp.bfloat16), jnp.uint16)
    packed = lo.astype(jnp.uint32) | (hi.astype(jnp.uint32) << 16)
    uP_ref[...] = lax.bitcast_convert_type(packed, jnp.int32)


def _node_transform(xT, posT, Wx, Wp, b2, NP, D, BN):
    grid = (NP // BN,)
    return pl.pallas_call(
        _tc_node_body,
        grid=grid,
        in_specs=[
            pl.BlockSpec((D, BN), lambda j: (0, j)),
            pl.BlockSpec((2, BN), lambda j: (0, j)),
            pl.BlockSpec((D, D), lambda j: (0, 0)),
            pl.BlockSpec((D, 3), lambda j: (0, 0)),
            pl.BlockSpec((D, 1), lambda j: (0, 0)),
        ],
        out_specs=[
            pl.BlockSpec((D // 2, BN), lambda j: (0, j)),
            pl.BlockSpec((D, BN), lambda j: (0, j)),
            pl.BlockSpec((3, BN), lambda j: (0, j)),
        ],
        out_shape=[
            jax.ShapeDtypeStruct((D // 2, NP), jnp.int32),
            jax.ShapeDtypeStruct((D, NP), jnp.float32),
            jax.ShapeDtypeStruct((3, NP), jnp.float32),
        ],
    )(xT, posT, Wx, Wp, b2)


def _make_sc_agg(D, NP, E, CH, NC, NS):
    NW = NC * NS
    H = D // 2
    PPT = H // NW
    mesh = plsc.VectorSubcoreMesh(
        core_axis_name="c", subcore_axis_name="s", num_cores=NC, num_subcores=NS
    )

    @functools.partial(
        pl.kernel,
        out_type=jax.ShapeDtypeStruct((D * NP,), jnp.float32),
        mesh=mesh,
        compiler_params=pltpu.CompilerParams(needs_layout_passes=False),
        scratch_types=[
            pltpu.VMEM((PPT * NP,), jnp.int32),
            pltpu.VMEM((PPT * NP,), jnp.int32),
            pltpu.VMEM((2 * PPT * NP,), jnp.float32),
            pltpu.VMEM((CH,), jnp.int32),
            pltpu.VMEM((CH,), jnp.int32),
            pltpu.VMEM((CH,), jnp.int32),
            pltpu.VMEM((CH,), jnp.int32),
            pltpu.SemaphoreType.DMA,
            pltpu.SemaphoreType.DMA,
            pltpu.SemaphoreType.DMA,
            pltpu.SemaphoreType.DMA,
        ],
    )
    def agg(
        uP_hbm,
        vT_hbm,
        src_hbm,
        dst_hbm,
        out_hbm,
        u_v,
        acc_v,
        v_v,
        src_a,
        src_b,
        dst_a,
        dst_b,
        sem_sa,
        sem_sb,
        sem_da,
        sem_db,
    ):
        cid = lax.axis_index("c")
        sid = lax.axis_index("s")
        wid = sid * NC + cid
        pbase = wid * (PPT * NP)
        pltpu.sync_copy(uP_hbm.at[pl.ds(pbase, PPT * NP)], u_v)
        pltpu.sync_copy(uP_hbm.at[pl.ds(pbase, PPT * NP)], acc_v)
        pltpu.sync_copy(
            vT_hbm.at[pl.ds(pbase, PPT * NP)], v_v.at[pl.ds(0, PPT * NP)]
        )
        pltpu.sync_copy(
            vT_hbm.at[pl.ds(H * NP + pbase, PPT * NP)],
            v_v.at[pl.ds(PPT * NP, PPT * NP)],
        )

        HIMASK = jnp.int32(-65536)

        def _unpk(w):
            lo = plsc.bitcast(jnp.left_shift(w, 16), jnp.float32)
            hi = plsc.bitcast(jnp.bitwise_and(w, HIMASK), jnp.float32)
            return lo, hi

        def _pk(lo, hi):
            lw = lax.shift_right_logical(plsc.bitcast(lo, jnp.int32), 16)
            hw = jnp.bitwise_and(plsc.bitcast(hi, jnp.int32), HIMASK)
            return jnp.bitwise_or(lw, hw)

        def process(src_v, dst_v):
            def vec_body(i, _):
                s16 = src_v[pl.ds(i * _LANES, _LANES)]
                d16 = dst_v[pl.ds(i * _LANES, _LANES)]
                idx_s = [s16 + jnp.int32(p * NP) for p in range(PPT)]
                idx_d = [d16 + jnp.int32(p * NP) for p in range(PPT)]
                gw = [plsc.load_gather(u_v, [idx_s[p]]) for p in range(PPT)]
                aw = [plsc.load_gather(acc_v, [idx_d[p]]) for p in range(PPT)]
                m = []
                for p in range(PPT):
                    g_lo, g_hi = _unpk(gw[p])
                    a_lo, a_hi = _unpk(aw[p])
                    m.append((jnp.maximum(g_lo, a_lo), jnp.maximum(g_hi, a_hi)))
                for p in range(PPT):
                    plsc.store_scatter(acc_v, [idx_d[p]], _pk(*m[p]))
                rb = [
                    _unpk(plsc.load_gather(acc_v, [idx_d[p]])) for p in range(PPT)
                ]
                bad = (m[0][0] > rb[0][0]) | (m[0][1] > rb[0][1])
                for p in range(1, PPT):
                    bad = bad | (m[p][0] > rb[p][0]) | (m[p][1] > rb[p][1])

                @pl.when(jnp.any(bad))
                def _fixup():
                    for p in range(PPT):

                        def conflict_cond(carry):
                            ml, mh, rl, rh = carry
                            return jnp.any((ml > rl) | (mh > rh))

                        def conflict_body(carry, p=p):
                            ml, mh, rl, rh = carry
                            wmask = (ml > rl) | (mh > rh)
                            plsc.store_scatter(
                                acc_v, [idx_d[p]], _pk(ml, mh), mask=wmask
                            )
                            rl2, rh2 = _unpk(
                                plsc.load_gather(acc_v, [idx_d[p]])
                            )
                            return (
                                jnp.maximum(ml, rl2),
                                jnp.maximum(mh, rh2),
                                rl2,
                                rh2,
                            )

                        lax.while_loop(
                            conflict_cond,
                            conflict_body,
                            (m[p][0], m[p][1], rb[p][0], rb[p][1]),
                        )
                return 0

            lax.fori_loop(0, CH // _LANES, vec_body, 0)

        nchunks = E // CH
        bufs = [(src_a, dst_a, sem_sa, sem_da), (src_b, dst_b, sem_sb, sem_db)]
        for bi, (sb, db, ss, sd) in enumerate(bufs):
            pltpu.async_copy(src_hbm.at[pl.ds(bi * CH, CH)], sb, ss)
            pltpu.async_copy(dst_hbm.at[pl.ds(bi * CH, CH)], db, sd)

        def outer_body(oi, _):
            for bi, (sb, db, ss, sd) in enumerate(bufs):
                ci = oi * 2 + bi
                pltpu.make_async_copy(src_hbm.at[pl.ds(ci * CH, CH)], sb, ss).wait()
                pltpu.make_async_copy(dst_hbm.at[pl.ds(ci * CH, CH)], db, sd).wait()
                process(sb, db)
                nci = lax.rem(ci + 2, nchunks)
                pltpu.async_copy(src_hbm.at[pl.ds(nci * CH, CH)], sb, ss)
                pltpu.async_copy(dst_hbm.at[pl.ds(nci * CH, CH)], db, sd)
            return 0

        lax.fori_loop(0, nchunks // 2, outer_body, 0)
        for bi, (sb, db, ss, sd) in enumerate(bufs):
            pltpu.make_async_copy(src_hbm.at[pl.ds(bi * CH, CH)], sb, ss).wait()
            pltpu.make_async_copy(dst_hbm.at[pl.ds(bi * CH, CH)], db, sd).wait()

        for p in range(PPT):

            def sub_body(i, _, p=p):
                sl = pl.ds(p * NP + i * _LANES, _LANES)
                slh = pl.ds((PPT + p) * NP + i * _LANES, _LANES)
                lo, hi = _unpk(acc_v[sl])
                v_v[sl] = lo - v_v[sl]
                v_v[slh] = hi - v_v[slh]
                return 0

            lax.fori_loop(0, NP // _LANES, sub_body, 0)
        pltpu.sync_copy(
            v_v.at[pl.ds(0, PPT * NP)], out_hbm.at[pl.ds(pbase, PPT * NP)]
        )
        pltpu.sync_copy(
            v_v.at[pl.ds(PPT * NP, PPT * NP)],
            out_hbm.at[pl.ds(H * NP + pbase, PPT * NP)],
        )

    return agg


def kernel(x, pos, edge_index, batch, W, b):
    N, D = x.shape
    E = edge_index.shape[1]
    NC, NS = 2, 16
    NW = NC * NS
    assert (D // 2) % NW == 0
    NP = -(-N // 256) * 256

    CH = 0
    for cand in range(4096, 15, -16):
        if E % cand == 0 and (E // cand) % 2 == 0:
            CH = cand
            break
    assert CH > 0

    xT = jnp.pad(x.T, ((0, 0), (0, NP - N)))
    posT = jnp.pad(pos.T, ((0, 0), (0, NP - N)))
    Wx = W[:, :D]
    Wp = W[:, D:]
    b2 = b[:, None]

    uP, vT, p3T = _node_transform(xT, posT, Wx, Wp, b2, NP, D, 512)

    src = edge_index[0]
    dst = edge_index[1]
    agg = _make_sc_agg(D, NP, E, CH, NC, NS)
    outF = agg(uP.reshape((D // 2) * NP), vT.reshape(D * NP), src, dst)

    out = outF.reshape(D, NP)[:, :N].T
    pos3 = p3T[:, :N].T
    return (out, pos3, batch)


# speedup vs baseline: 6.9264x; 1.5643x over previous
import functools

import jax
import jax.numpy as jnp
from jax import lax
from jax.experimental import pallas as pl
from jax.experimental.pallas import tpu as pltpu
from jax.experimental.pallas import tpu_sc as plsc

_LANES = 16


def _tc_node_body(xT_ref, posT_ref, Wx_ref, Wp_ref, b_ref, uP_ref, vT_ref, p3T_ref):
    pz = posT_ref[0:1, :]
    phi = posT_ref[1:2, :]
    px = jnp.cos(phi)
    py = jnp.sin(phi)
    p3T_ref[...] = jnp.concatenate([px, py, pz], axis=0)
    Wp = Wp_ref[...]
    v = Wp[:, 0:1] * px + Wp[:, 1:2] * py + Wp[:, 2:3] * pz
    vT_ref[...] = v
    u = (
        jnp.dot(Wx_ref[...], xT_ref[...], preferred_element_type=jnp.float32)
        + v
        + b_ref[...]
    )
    D = u.shape[0]
    H = D // 2
    lo = lax.bitcast_convert_type(u[:H].astype(jnp.bfloat16), jnp.uint16)
    hi = lax.bitcast_convert_type(u[H:].astype(jnp.bfloat16), jnp.uint16)
    packed = lo.astype(jnp.uint32) | (hi.astype(jnp.uint32) << 16)
    uP_ref[...] = lax.bitcast_convert_type(packed, jnp.int32)


def _node_transform(xT, posT, Wx, Wp, b2, NP, D, BN):
    grid = (NP // BN,)
    return pl.pallas_call(
        _tc_node_body,
        grid=grid,
        in_specs=[
            pl.BlockSpec((D, BN), lambda j: (0, j)),
            pl.BlockSpec((2, BN), lambda j: (0, j)),
            pl.BlockSpec((D, D), lambda j: (0, 0)),
            pl.BlockSpec((D, 3), lambda j: (0, 0)),
            pl.BlockSpec((D, 1), lambda j: (0, 0)),
        ],
        out_specs=[
            pl.BlockSpec((D // 2, BN), lambda j: (0, j)),
            pl.BlockSpec((D, BN), lambda j: (0, j)),
            pl.BlockSpec((3, BN), lambda j: (0, j)),
        ],
        out_shape=[
            jax.ShapeDtypeStruct((D // 2, NP), jnp.int32),
            jax.ShapeDtypeStruct((D, NP), jnp.float32),
            jax.ShapeDtypeStruct((3, NP), jnp.float32),
        ],
    )(xT, posT, Wx, Wp, b2)


def _make_sc_agg(D, NP, E, CH, NC, NS):
    NW = NC * NS
    H = D // 2
    PPT = H // NW
    mesh = plsc.VectorSubcoreMesh(
        core_axis_name="c", subcore_axis_name="s", num_cores=NC, num_subcores=NS
    )

    @functools.partial(
        pl.kernel,
        out_type=jax.ShapeDtypeStruct((D * NP,), jnp.float32),
        mesh=mesh,
        compiler_params=pltpu.CompilerParams(needs_layout_passes=False),
        scratch_types=[
            pltpu.VMEM((PPT * NP,), jnp.int32),
            pltpu.VMEM((PPT * NP,), jnp.int32),
            pltpu.VMEM((2 * PPT * NP,), jnp.float32),
            pltpu.VMEM((CH,), jnp.int32),
            pltpu.VMEM((CH,), jnp.int32),
            pltpu.VMEM((CH,), jnp.int32),
            pltpu.VMEM((CH,), jnp.int32),
            pltpu.SemaphoreType.DMA,
            pltpu.SemaphoreType.DMA,
            pltpu.SemaphoreType.DMA,
            pltpu.SemaphoreType.DMA,
        ],
    )
    def agg(
        uP_hbm,
        vT_hbm,
        src_hbm,
        dst_hbm,
        out_hbm,
        u_v,
        acc_v,
        v_v,
        src_a,
        src_b,
        dst_a,
        dst_b,
        sem_sa,
        sem_sb,
        sem_da,
        sem_db,
    ):
        cid = lax.axis_index("c")
        sid = lax.axis_index("s")
        wid = sid * NC + cid
        pbase = wid * (PPT * NP)
        pltpu.sync_copy(uP_hbm.at[pl.ds(pbase, PPT * NP)], u_v)
        pltpu.sync_copy(uP_hbm.at[pl.ds(pbase, PPT * NP)], acc_v)
        pltpu.sync_copy(
            vT_hbm.at[pl.ds(pbase, PPT * NP)], v_v.at[pl.ds(0, PPT * NP)]
        )
        pltpu.sync_copy(
            vT_hbm.at[pl.ds(H * NP + pbase, PPT * NP)],
            v_v.at[pl.ds(PPT * NP, PPT * NP)],
        )

        HIMASK = jnp.int32(-65536)

        def _unpk(w):
            lo = plsc.bitcast(jnp.left_shift(w, 16), jnp.float32)
            hi = plsc.bitcast(jnp.bitwise_and(w, HIMASK), jnp.float32)
            return lo, hi

        def _pk(lo, hi):
            lw = lax.shift_right_logical(plsc.bitcast(lo, jnp.int32), 16)
            hw = jnp.bitwise_and(plsc.bitcast(hi, jnp.int32), HIMASK)
            return jnp.bitwise_or(lw, hw)

        U = 2

        def process(src_v, dst_v):
            def vec_body(i, _):
                idx_d, gw, aw = [], [], []
                for k in range(U):
                    base = (i * U + k) * _LANES
                    s16 = src_v[pl.ds(base, _LANES)]
                    d16 = dst_v[pl.ds(base, _LANES)]
                    for p in range(PPT):
                        idx_d.append(d16 + jnp.int32(p * NP))
                        gw.append(plsc.load_gather(u_v, [s16 + jnp.int32(p * NP)]))
                for j in range(U * PPT):
                    aw.append(plsc.load_gather(acc_v, [idx_d[j]]))
                m = []
                for j in range(U * PPT):
                    g_lo, g_hi = _unpk(gw[j])
                    a_lo, a_hi = _unpk(aw[j])
                    m.append((jnp.maximum(g_lo, a_lo), jnp.maximum(g_hi, a_hi)))
                for j in range(U * PPT):
                    plsc.store_scatter(acc_v, [idx_d[j]], _pk(*m[j]))
                rb = [
                    _unpk(plsc.load_gather(acc_v, [idx_d[j]]))
                    for j in range(U * PPT)
                ]
                bad = (m[0][0] > rb[0][0]) | (m[0][1] > rb[0][1])
                for j in range(1, U * PPT):
                    bad = bad | (m[j][0] > rb[j][0]) | (m[j][1] > rb[j][1])

                def fix_cond(dirty):
                    return dirty

                def fix_body(_):
                    dirty = jnp.bool_(False)
                    for j in range(U * PPT):
                        rl, rh = _unpk(plsc.load_gather(acc_v, [idx_d[j]]))
                        need = (m[j][0] > rl) | (m[j][1] > rh)
                        wl = jnp.maximum(m[j][0], rl)
                        wh = jnp.maximum(m[j][1], rh)
                        plsc.store_scatter(
                            acc_v, [idx_d[j]], _pk(wl, wh), mask=need
                        )
                        dirty = dirty | jnp.any(need)
                    return dirty

                lax.while_loop(fix_cond, fix_body, jnp.any(bad))
                return 0

            lax.fori_loop(0, CH // (U * _LANES), vec_body, 0)

        nchunks = E // CH
        bufs = [(src_a, dst_a, sem_sa, sem_da), (src_b, dst_b, sem_sb, sem_db)]
        for bi, (sb, db, ss, sd) in enumerate(bufs):
            pltpu.async_copy(src_hbm.at[pl.ds(bi * CH, CH)], sb, ss)
            pltpu.async_copy(dst_hbm.at[pl.ds(bi * CH, CH)], db, sd)

        def outer_body(oi, _):
            for bi, (sb, db, ss, sd) in enumerate(bufs):
                ci = oi * 2 + bi
                pltpu.make_async_copy(src_hbm.at[pl.ds(ci * CH, CH)], sb, ss).wait()
                pltpu.make_async_copy(dst_hbm.at[pl.ds(ci * CH, CH)], db, sd).wait()
                process(sb, db)
                nci = lax.rem(ci + 2, nchunks)
                pltpu.async_copy(src_hbm.at[pl.ds(nci * CH, CH)], sb, ss)
                pltpu.async_copy(dst_hbm.at[pl.ds(nci * CH, CH)], db, sd)
            return 0

        lax.fori_loop(0, nchunks // 2, outer_body, 0)
        for bi, (sb, db, ss, sd) in enumerate(bufs):
            pltpu.make_async_copy(src_hbm.at[pl.ds(bi * CH, CH)], sb, ss).wait()
            pltpu.make_async_copy(dst_hbm.at[pl.ds(bi * CH, CH)], db, sd).wait()

        for p in range(PPT):

            def sub_body(i, _, p=p):
                sl = pl.ds(p * NP + i * _LANES, _LANES)
                slh = pl.ds((PPT + p) * NP + i * _LANES, _LANES)
                lo, hi = _unpk(acc_v[sl])
                v_v[sl] = lo - v_v[sl]
                v_v[slh] = hi - v_v[slh]
                return 0

            lax.fori_loop(0, NP // _LANES, sub_body, 0)
        pltpu.sync_copy(
            v_v.at[pl.ds(0, PPT * NP)], out_hbm.at[pl.ds(pbase, PPT * NP)]
        )
        pltpu.sync_copy(
            v_v.at[pl.ds(PPT * NP, PPT * NP)],
            out_hbm.at[pl.ds(H * NP + pbase, PPT * NP)],
        )

    return agg


def kernel(x, pos, edge_index, batch, W, b):
    N, D = x.shape
    E = edge_index.shape[1]
    NC, NS = 2, 16
    NW = NC * NS
    assert (D // 2) % NW == 0
    NP = -(-N // 256) * 256

    CH = 0
    for cand in range(4096, 31, -32):
        if E % cand == 0 and (E // cand) % 2 == 0:
            CH = cand
            break
    assert CH > 0

    xT = jnp.pad(x.T, ((0, 0), (0, NP - N)))
    posT = jnp.pad(pos.T, ((0, 0), (0, NP - N)))
    Wx = W[:, :D]
    Wp = W[:, D:]
    b2 = b[:, None]

    uP, vT, p3T = _node_transform(xT, posT, Wx, Wp, b2, NP, D, 512)

    src = edge_index[0]
    dst = edge_index[1]
    agg = _make_sc_agg(D, NP, E, CH, NC, NS)
    outF = agg(uP.reshape((D // 2) * NP), vT.reshape(D * NP), src, dst)

    out = outF.reshape(D, NP)[:, :N].T
    pos3 = p3T[:, :N].T
    return (out, pos3, batch)


# speedup vs baseline: 8.9771x; 1.2961x over previous
import functools

import jax
import jax.numpy as jnp
from jax import lax
from jax.experimental import pallas as pl
from jax.experimental.pallas import tpu as pltpu
from jax.experimental.pallas import tpu_sc as plsc

_LANES = 16


def _tc_node_body(xT_ref, posT_ref, Wx_ref, Wp_ref, b_ref, uP_ref, vT_ref, p3T_ref):
    pz = posT_ref[0:1, :]
    phi = posT_ref[1:2, :]
    px = jnp.cos(phi)
    py = jnp.sin(phi)
    p3T_ref[...] = jnp.concatenate([px, py, pz], axis=0)
    Wp = Wp_ref[...]
    v = Wp[:, 0:1] * px + Wp[:, 1:2] * py + Wp[:, 2:3] * pz
    vT_ref[...] = v
    u = (
        jnp.dot(Wx_ref[...], xT_ref[...], preferred_element_type=jnp.float32)
        + v
        + b_ref[...]
    )
    D = u.shape[0]
    H = D // 2
    lo = lax.bitcast_convert_type(u[:H].astype(jnp.bfloat16), jnp.uint16)
    hi = lax.bitcast_convert_type(u[H:].astype(jnp.bfloat16), jnp.uint16)
    packed = lo.astype(jnp.uint32) | (hi.astype(jnp.uint32) << 16)
    uP_ref[...] = lax.bitcast_convert_type(packed, jnp.int32)


def _node_transform(xT, posT, Wx, Wp, b2, NP, D, BN):
    grid = (NP // BN,)
    return pl.pallas_call(
        _tc_node_body,
        grid=grid,
        in_specs=[
            pl.BlockSpec((D, BN), lambda j: (0, j)),
            pl.BlockSpec((2, BN), lambda j: (0, j)),
            pl.BlockSpec((D, D), lambda j: (0, 0)),
            pl.BlockSpec((D, 3), lambda j: (0, 0)),
            pl.BlockSpec((D, 1), lambda j: (0, 0)),
        ],
        out_specs=[
            pl.BlockSpec((D // 2, BN), lambda j: (0, j)),
            pl.BlockSpec((D, BN), lambda j: (0, j)),
            pl.BlockSpec((3, BN), lambda j: (0, j)),
        ],
        out_shape=[
            jax.ShapeDtypeStruct((D // 2, NP), jnp.int32),
            jax.ShapeDtypeStruct((D, NP), jnp.float32),
            jax.ShapeDtypeStruct((3, NP), jnp.float32),
        ],
    )(xT, posT, Wx, Wp, b2)


def _make_sc_agg(D, NP, E, CH, NC, NS):
    NW = NC * NS
    H = D // 2
    PPT = H // NW
    mesh = plsc.VectorSubcoreMesh(
        core_axis_name="c", subcore_axis_name="s", num_cores=NC, num_subcores=NS
    )

    @functools.partial(
        pl.kernel,
        out_type=jax.ShapeDtypeStruct((D * NP,), jnp.float32),
        mesh=mesh,
        compiler_params=pltpu.CompilerParams(needs_layout_passes=False),
        scratch_types=[
            pltpu.VMEM((PPT * NP,), jnp.int32),
            pltpu.VMEM((PPT * NP,), jnp.int32),
            pltpu.VMEM((2 * PPT * NP,), jnp.float32),
            pltpu.VMEM((CH,), jnp.int32),
            pltpu.VMEM((CH,), jnp.int32),
            pltpu.VMEM((CH,), jnp.int32),
            pltpu.VMEM((CH,), jnp.int32),
            pltpu.SemaphoreType.DMA,
            pltpu.SemaphoreType.DMA,
            pltpu.SemaphoreType.DMA,
            pltpu.SemaphoreType.DMA,
        ],
    )
    def agg(
        uP_hbm,
        vT_hbm,
        src_hbm,
        dst_hbm,
        out_hbm,
        u_v,
        acc_v,
        v_v,
        src_a,
        src_b,
        dst_a,
        dst_b,
        sem_sa,
        sem_sb,
        sem_da,
        sem_db,
    ):
        cid = lax.axis_index("c")
        sid = lax.axis_index("s")
        wid = sid * NC + cid
        pbase = wid * (PPT * NP)
        pltpu.sync_copy(uP_hbm.at[pl.ds(pbase, PPT * NP)], u_v)
        pltpu.sync_copy(uP_hbm.at[pl.ds(pbase, PPT * NP)], acc_v)
        pltpu.sync_copy(
            vT_hbm.at[pl.ds(pbase, PPT * NP)], v_v.at[pl.ds(0, PPT * NP)]
        )
        pltpu.sync_copy(
            vT_hbm.at[pl.ds(H * NP + pbase, PPT * NP)],
            v_v.at[pl.ds(PPT * NP, PPT * NP)],
        )

        HIMASK = jnp.int32(-65536)

        def _unpk(w):
            lo = plsc.bitcast(jnp.left_shift(w, 16), jnp.float32)
            hi = plsc.bitcast(jnp.bitwise_and(w, HIMASK), jnp.float32)
            return lo, hi

        def _pk(lo, hi):
            lw = lax.shift_right_logical(plsc.bitcast(lo, jnp.int32), 16)
            hw = jnp.bitwise_and(plsc.bitcast(hi, jnp.int32), HIMASK)
            return jnp.bitwise_or(lw, hw)

        U = 4

        def process(src_v, dst_v):
            def vec_body(i, _):
                idx_d, gw, aw = [], [], []
                for k in range(U):
                    base = (i * U + k) * _LANES
                    s16 = src_v[pl.ds(base, _LANES)]
                    d16 = dst_v[pl.ds(base, _LANES)]
                    for p in range(PPT):
                        idx_d.append(d16 + jnp.int32(p * NP))
                        gw.append(plsc.load_gather(u_v, [s16 + jnp.int32(p * NP)]))
                for j in range(U * PPT):
                    aw.append(plsc.load_gather(acc_v, [idx_d[j]]))
                m = []
                for j in range(U * PPT):
                    g_lo, g_hi = _unpk(gw[j])
                    a_lo, a_hi = _unpk(aw[j])
                    m.append((jnp.maximum(g_lo, a_lo), jnp.maximum(g_hi, a_hi)))
                for j in range(U * PPT):
                    plsc.store_scatter(acc_v, [idx_d[j]], _pk(*m[j]))
                rb = [
                    _unpk(plsc.load_gather(acc_v, [idx_d[j]]))
                    for j in range(U * PPT)
                ]
                bad = (m[0][0] > rb[0][0]) | (m[0][1] > rb[0][1])
                for j in range(1, U * PPT):
                    bad = bad | (m[j][0] > rb[j][0]) | (m[j][1] > rb[j][1])

                def fix_cond(dirty):
                    return dirty

                def fix_body(_):
                    dirty = jnp.bool_(False)
                    for j in range(U * PPT):
                        rl, rh = _unpk(plsc.load_gather(acc_v, [idx_d[j]]))
                        need = (m[j][0] > rl) | (m[j][1] > rh)
                        wl = jnp.maximum(m[j][0], rl)
                        wh = jnp.maximum(m[j][1], rh)
                        plsc.store_scatter(
                            acc_v, [idx_d[j]], _pk(wl, wh), mask=need
                        )
                        dirty = dirty | jnp.any(need)
                    return dirty

                lax.while_loop(fix_cond, fix_body, jnp.any(bad))
                return 0

            lax.fori_loop(0, CH // (U * _LANES), vec_body, 0)

        nchunks = E // CH
        bufs = [(src_a, dst_a, sem_sa, sem_da), (src_b, dst_b, sem_sb, sem_db)]
        for bi, (sb, db, ss, sd) in enumerate(bufs):
            pltpu.async_copy(src_hbm.at[pl.ds(bi * CH, CH)], sb, ss)
            pltpu.async_copy(dst_hbm.at[pl.ds(bi * CH, CH)], db, sd)

        def outer_body(oi, _):
            for bi, (sb, db, ss, sd) in enumerate(bufs):
                ci = oi * 2 + bi
                pltpu.make_async_copy(src_hbm.at[pl.ds(ci * CH, CH)], sb, ss).wait()
                pltpu.make_async_copy(dst_hbm.at[pl.ds(ci * CH, CH)], db, sd).wait()
                process(sb, db)
                nci = lax.rem(ci + 2, nchunks)
                pltpu.async_copy(src_hbm.at[pl.ds(nci * CH, CH)], sb, ss)
                pltpu.async_copy(dst_hbm.at[pl.ds(nci * CH, CH)], db, sd)
            return 0

        lax.fori_loop(0, nchunks // 2, outer_body, 0)
        for bi, (sb, db, ss, sd) in enumerate(bufs):
            pltpu.make_async_copy(src_hbm.at[pl.ds(bi * CH, CH)], sb, ss).wait()
            pltpu.make_async_copy(dst_hbm.at[pl.ds(bi * CH, CH)], db, sd).wait()

        for p in range(PPT):

            def sub_body(i, _, p=p):
                sl = pl.ds(p * NP + i * _LANES, _LANES)
                slh = pl.ds((PPT + p) * NP + i * _LANES, _LANES)
                lo, hi = _unpk(acc_v[sl])
                v_v[sl] = lo - v_v[sl]
                v_v[slh] = hi - v_v[slh]
                return 0

            lax.fori_loop(0, NP // _LANES, sub_body, 0)
        pltpu.sync_copy(
            v_v.at[pl.ds(0, PPT * NP)], out_hbm.at[pl.ds(pbase, PPT * NP)]
        )
        pltpu.sync_copy(
            v_v.at[pl.ds(PPT * NP, PPT * NP)],
            out_hbm.at[pl.ds(H * NP + pbase, PPT * NP)],
        )

    return agg


def kernel(x, pos, edge_index, batch, W, b):
    N, D = x.shape
    E = edge_index.shape[1]
    NC, NS = 2, 16
    NW = NC * NS
    assert (D // 2) % NW == 0
    NP = -(-N // 256) * 256

    CH = 0
    for cand in range(4096, 63, -64):
        if E % cand == 0 and (E // cand) % 2 == 0:
            CH = cand
            break
    assert CH > 0

    xT = jnp.pad(x.T, ((0, 0), (0, NP - N)))
    posT = jnp.pad(pos.T, ((0, 0), (0, NP - N)))
    Wx = W[:, :D]
    Wp = W[:, D:]
    b2 = b[:, None]

    uP, vT, p3T = _node_transform(xT, posT, Wx, Wp, b2, NP, D, 512)

    src = edge_index[0]
    dst = edge_index[1]
    agg = _make_sc_agg(D, NP, E, CH, NC, NS)
    outF = agg(uP.reshape((D // 2) * NP), vT.reshape(D * NP), src, dst)

    out = outF.reshape(D, NP)[:, :N].T
    pos3 = p3T[:, :N].T
    return (out, pos3, batch)
